# baseline (speedup 1.0000x reference)
import functools

import jax
import jax.numpy as jnp
from jax import lax
from jax.experimental import pallas as pl
from jax.experimental.pallas import tpu as pltpu
from jax.experimental.pallas import tpu_sc as plsc

NC = 2
NS = 16
NW = NC * NS
LANES = 16
NBUF = 2


@functools.partial(jax.jit, static_argnames=("B", "L", "D"))
def _embed(ids_t, tab_pad, pos_flat, B, L, D):
    scale = float(D) ** 0.5
    LB = L // 8
    BT = B // 128
    NTILES = LB * BT
    TPW = NTILES // NW
    CPW = TPW * 8
    G = D // LANES

    mesh = plsc.VectorSubcoreMesh(core_axis_name="c", subcore_axis_name="s")

    @functools.partial(
        pl.kernel,
        mesh=mesh,
        out_type=jax.ShapeDtypeStruct((L, B, D), jnp.float32),
        compiler_params=pltpu.CompilerParams(needs_layout_passes=False),
        scratch_types=(
            [pltpu.VMEM((8, 128), jnp.int32)]
            + [pltpu.VMEM((128,), jnp.int32)] * NBUF
            + [pltpu.VMEM((128, 2 * D), jnp.float32)] * NBUF
            + [pltpu.VMEM((128, D), jnp.float32)] * NBUF
            + [pltpu.VMEM((L * D,), jnp.float32)]
            + [pltpu.SemaphoreType.DMA] * NBUF
            + [pltpu.SemaphoreType.DMA] * NBUF
        ),
    )
    def k(ids_hbm, tab_hbm, pos_hbm, out_hbm, idx_tile, *sc):
        idx2 = sc[0:NBUF]
        rows = sc[NBUF:2 * NBUF]
        st = sc[2 * NBUF:3 * NBUF]
        pos_v = sc[3 * NBUF]
        gsems = sc[3 * NBUF + 1:3 * NBUF + 1 + NBUF]
        osems = sc[3 * NBUF + 1 + NBUF:3 * NBUF + 1 + 2 * NBUF]

        wid = lax.axis_index("s") * NC + lax.axis_index("c")
        t0 = wid * TPW

        pltpu.sync_copy(pos_hbm, pos_v)

        def load_tile(gg):
            t = t0 + gg // 8
            lb = t // BT
            bt = t % BT
            pltpu.sync_copy(
                ids_hbm.at[pl.ds(lb * 8, 8), pl.ds(bt * 128, 128)], idx_tile)

        def build_and_start(b, gg):
            l8 = gg % 8
            for g in range(128 // LANES):
                idx2[b][pl.ds(g * LANES, LANES)] = (
                    idx_tile[l8, pl.ds(g * LANES, LANES)])
            pltpu.make_async_copy(
                tab_hbm.at[idx2[b]], rows[b], gsems[b]).start()

        def out_slice(gg):
            t = t0 + gg // 8
            lb = t // BT
            bt = t % BT
            l = lb * 8 + gg % 8
            return out_hbm.at[l, pl.ds(bt * 128, 128), :], l

        load_tile(0)
        for b in range(NBUF):
            build_and_start(b, b)

        def chunk_body(g2, _):
            for b in range(NBUF):
                gg = g2 * NBUF + b
                pltpu.make_async_copy(
                    tab_hbm.at[idx2[b]], rows[b], gsems[b]).wait()

                dst, l = out_slice(gg)

                @pl.when(gg >= NBUF)
                def _():
                    pltpu.make_async_copy(st[b], dst, osems[b]).wait()

                pvq = [pos_v[pl.ds(l * D + q * LANES, LANES)]
                       for q in range(G)]

                @plsc.parallel_loop(0, 128, 1, unroll=2)
                def _(r):
                    for q in range(G):
                        x = rows[b][r, pl.ds(q * LANES, LANES)]
                        st[b][r, pl.ds(q * LANES, LANES)] = x + pvq[q]

                pltpu.make_async_copy(st[b], dst, osems[b]).start()

                @pl.when(gg + NBUF < CPW)
                def _():
                    gg2 = gg + NBUF

                    @pl.when(gg2 % 8 == 0)
                    def _():
                        load_tile(gg2)

                    build_and_start(b, gg2)
            return 0

        lax.fori_loop(0, CPW // NBUF, chunk_body, 0)

        for b in range(NBUF):
            dst, _ = out_slice(CPW - NBUF + b)
            pltpu.make_async_copy(st[b], dst, osems[b]).wait()

    return k(ids_t, tab_pad, pos_flat)


def kernel(input_word_ids, token_table, position_table):
    B, L = input_word_ids.shape
    V, D = token_table.shape
    ids_t = jnp.swapaxes(input_word_ids, 0, 1).astype(jnp.int32)
    tab_pad = jnp.pad(token_table * jnp.float32(float(D) ** 0.5),
                      ((0, 0), (0, D)))
    pos_flat = position_table.reshape(L * D)
    out3 = _embed(ids_t, tab_pad, pos_flat, B, L, D)
    return jnp.transpose(out3, (1, 0, 2))


# speedup vs baseline: 1.3185x; 1.3185x over previous
import functools

import jax
import jax.numpy as jnp
from jax import lax
from jax.experimental import pallas as pl
from jax.experimental.pallas import tpu as pltpu
from jax.experimental.pallas import tpu_sc as plsc

NC = 2
NS = 16
NW = NC * NS
LANES = 16
NBUF = 2


@functools.partial(jax.jit, static_argnames=("B", "L", "D"))
def _embed(ids_t, tab_pad, pos_flat, B, L, D):
    scale = float(D) ** 0.5
    LB = L // 8
    BT = B // 128
    NTILES = LB * BT
    TPW = NTILES // NW
    CPW = TPW * 8
    G = D // LANES

    mesh = plsc.VectorSubcoreMesh(core_axis_name="c", subcore_axis_name="s")

    @functools.partial(
        pl.kernel,
        mesh=mesh,
        out_type=jax.ShapeDtypeStruct((L, B, D), jnp.float32),
        compiler_params=pltpu.CompilerParams(needs_layout_passes=False),
        scratch_types=(
            [pltpu.VMEM((8, 128), jnp.int32)]
            + [pltpu.VMEM((128,), jnp.int32)] * NBUF
            + [pltpu.VMEM((128, 2 * D), jnp.float32)] * NBUF
            + [pltpu.VMEM((128, D), jnp.float32)] * NBUF
            + [pltpu.VMEM((L * D,), jnp.float32)]
            + [pltpu.SemaphoreType.DMA] * NBUF
            + [pltpu.SemaphoreType.DMA] * NBUF
        ),
    )
    def k(ids_hbm, tab_hbm, pos_hbm, out_hbm, idx_tile, *sc):
        idx2 = sc[0:NBUF]
        rows = sc[NBUF:2 * NBUF]
        st = sc[2 * NBUF:3 * NBUF]
        pos_v = sc[3 * NBUF]
        gsems = sc[3 * NBUF + 1:3 * NBUF + 1 + NBUF]
        osems = sc[3 * NBUF + 1 + NBUF:3 * NBUF + 1 + 2 * NBUF]

        wid = lax.axis_index("s") * NC + lax.axis_index("c")
        t0 = wid * TPW

        pltpu.sync_copy(pos_hbm, pos_v)

        def load_tile(gg):
            t = t0 + gg // 8
            lb = t // BT
            bt = t % BT
            pltpu.sync_copy(
                ids_hbm.at[pl.ds(lb * 8, 8), pl.ds(bt * 128, 128)], idx_tile)

        def build_and_start(b, gg):
            l8 = gg % 8
            for g in range(128 // LANES):
                idx2[b][pl.ds(g * LANES, LANES)] = (
                    idx_tile[l8, pl.ds(g * LANES, LANES)])
            pltpu.make_async_copy(
                tab_hbm.at[idx2[b]], rows[b], gsems[b]).start()

        def out_slice(gg):
            t = t0 + gg // 8
            lb = t // BT
            bt = t % BT
            l = lb * 8 + gg % 8
            return out_hbm.at[l, pl.ds(bt * 128, 128), :], l

        load_tile(0)
        for b in range(NBUF):
            build_and_start(b, b)

        def chunk_body(g2, _):
            for b in range(NBUF):
                gg = g2 * NBUF + b
                pltpu.make_async_copy(
                    tab_hbm.at[idx2[b]], rows[b], gsems[b]).wait()

                dst, l = out_slice(gg)

                @pl.when(gg >= NBUF)
                def _():
                    pltpu.make_async_copy(st[b], dst, osems[b]).wait()

                pvq = [pos_v[pl.ds(l * D + q * LANES, LANES)]
                       for q in range(G)]

                @plsc.parallel_loop(0, 128, 1, unroll=4)
                def _(r):
                    for q in range(G):
                        x = rows[b][r, pl.ds(q * LANES, LANES)]
                        st[b][r, pl.ds(q * LANES, LANES)] = (
                            x * scale + pvq[q])

                pltpu.make_async_copy(st[b], dst, osems[b]).start()

                @pl.when(gg + NBUF < CPW)
                def _():
                    gg2 = gg + NBUF

                    @pl.when(gg2 % 8 == 0)
                    def _():
                        load_tile(gg2)

                    build_and_start(b, gg2)
            return 0

        lax.fori_loop(0, CPW // NBUF, chunk_body, 0)

        for b in range(NBUF):
            dst, _ = out_slice(CPW - NBUF + b)
            pltpu.make_async_copy(st[b], dst, osems[b]).wait()

    return k(ids_t, tab_pad, pos_flat)


def kernel(input_word_ids, token_table, position_table):
    B, L = input_word_ids.shape
    V, D = token_table.shape
    ids_t = jnp.swapaxes(input_word_ids, 0, 1).astype(jnp.int32)
    tab_pad = jnp.pad(token_table, ((0, 0), (0, D)))
    pos_flat = position_table.reshape(L * D)
    out3 = _embed(ids_t, tab_pad, pos_flat, B, L, D)
    return jnp.transpose(out3, (1, 0, 2))


# speedup vs baseline: 1.3208x; 1.0017x over previous
import functools

import jax
import jax.numpy as jnp
from jax import lax
from jax.experimental import pallas as pl
from jax.experimental.pallas import tpu as pltpu
from jax.experimental.pallas import tpu_sc as plsc

NC = 2
NS = 16
NW = NC * NS
LANES = 16
NBUF = 2


@functools.partial(jax.jit, static_argnames=("B", "L", "D"))
def _embed(ids_t, tab_pad, pos_flat, B, L, D):
    scale = float(D) ** 0.5
    LB = L // 8
    BT = B // 128
    NTILES = LB * BT
    TPW = NTILES // NW
    CPW = TPW * 8
    G = D // LANES

    mesh = plsc.VectorSubcoreMesh(core_axis_name="c", subcore_axis_name="s")

    @functools.partial(
        pl.kernel,
        mesh=mesh,
        out_type=jax.ShapeDtypeStruct((L, B, D), jnp.float32),
        compiler_params=pltpu.CompilerParams(needs_layout_passes=False),
        scratch_types=(
            [pltpu.VMEM((8, 128), jnp.int32)]
            + [pltpu.VMEM((128,), jnp.int32)] * NBUF
            + [pltpu.VMEM((128, 2 * D), jnp.float32)] * NBUF
            + [pltpu.VMEM((128, D), jnp.float32)] * NBUF
            + [pltpu.VMEM((L * D,), jnp.float32)]
            + [pltpu.SemaphoreType.DMA] * NBUF
            + [pltpu.SemaphoreType.DMA] * NBUF
        ),
    )
    def k(ids_hbm, tab_hbm, pos_hbm, out_hbm, idx_tile, *sc):
        idx2 = sc[0:NBUF]
        rows = sc[NBUF:2 * NBUF]
        st = sc[2 * NBUF:3 * NBUF]
        pos_v = sc[3 * NBUF]
        gsems = sc[3 * NBUF + 1:3 * NBUF + 1 + NBUF]
        osems = sc[3 * NBUF + 1 + NBUF:3 * NBUF + 1 + 2 * NBUF]

        wid = lax.axis_index("s") * NC + lax.axis_index("c")
        t0 = wid * TPW

        pltpu.sync_copy(pos_hbm, pos_v)

        def load_tile(gg):
            t = t0 + gg // 8
            lb = t // BT
            bt = t % BT
            pltpu.sync_copy(
                ids_hbm.at[pl.ds(lb * 8, 8), pl.ds(bt * 128, 128)], idx_tile)

        def build_and_start(b, gg):
            l8 = gg % 8
            for g in range(128 // LANES):
                idx2[b][pl.ds(g * LANES, LANES)] = (
                    idx_tile[l8, pl.ds(g * LANES, LANES)])
            pltpu.make_async_copy(
                tab_hbm.at[idx2[b]], rows[b], gsems[b]).start()

        def out_slice(gg):
            t = t0 + gg // 8
            lb = t // BT
            bt = t % BT
            l = lb * 8 + gg % 8
            return out_hbm.at[l, pl.ds(bt * 128, 128), :], l

        load_tile(0)
        for b in range(NBUF):
            build_and_start(b, b)

        def chunk_body(g2, _):
            for b in range(NBUF):
                gg = g2 * NBUF + b
                pltpu.make_async_copy(
                    tab_hbm.at[idx2[b]], rows[b], gsems[b]).wait()

                dst, l = out_slice(gg)

                @pl.when(gg >= NBUF)
                def _():
                    pltpu.make_async_copy(st[b], dst, osems[b]).wait()

                pvq = [pos_v[pl.ds(l * D + q * LANES, LANES)]
                       for q in range(G)]

                @plsc.parallel_loop(0, 128, 1, unroll=2)
                def _(r):
                    for q in range(G):
                        x = rows[b][r, pl.ds(q * LANES, LANES)]
                        st[b][r, pl.ds(q * LANES, LANES)] = (
                            x * scale + pvq[q])

                pltpu.make_async_copy(st[b], dst, osems[b]).start()

                @pl.when(gg + NBUF < CPW)
                def _():
                    gg2 = gg + NBUF

                    @pl.when(gg2 % 8 == 0)
                    def _():
                        load_tile(gg2)

                    build_and_start(b, gg2)
            return 0

        lax.fori_loop(0, CPW // NBUF, chunk_body, 0)

        for b in range(NBUF):
            dst, _ = out_slice(CPW - NBUF + b)
            pltpu.make_async_copy(st[b], dst, osems[b]).wait()

    return k(ids_t, tab_pad, pos_flat)


def kernel(input_word_ids, token_table, position_table):
    B, L = input_word_ids.shape
    V, D = token_table.shape
    ids_t = jnp.swapaxes(input_word_ids, 0, 1).astype(jnp.int32)
    tab_pad = jnp.pad(token_table, ((0, 0), (0, D)))
    pos_flat = position_table.reshape(L * D)
    out3 = _embed(ids_t, tab_pad, pos_flat, B, L, D)
    return jnp.transpose(out3, (1, 0, 2))
